# Initial kernel scaffold; baseline (speedup 1.0000x reference)
import functools

import jax
import jax.numpy as jnp
from jax import lax
from jax.experimental import pallas as pl
from jax.experimental.pallas import tpu as pltpu
from jax.experimental.pallas import tpu_sc as plsc

_B = 16384
_F = 26
_D = 16
_IN = _F * _D

_NC = 2
_NS = 16
_NW = _NC * _NS
_TOT = _B * _F
_RPW = _TOT // _NW
_CHUNK = 128
_NCH = _RPW // _CHUNK


def _sc_gather_body(table_hbm, idx_hbm, out_hbm, idx_v, bufa, bufb, sema, semb):
    wid = lax.axis_index("s") * _NC + lax.axis_index("c")
    pltpu.sync_copy(idx_hbm.at[pl.ds(wid * _NCH, _NCH)], idx_v)
    out_base = wid * _RPW
    pltpu.async_copy(table_hbm.at[idx_v.at[0]], bufa, sema)

    def step(jj, carry):
        j0 = 2 * jj
        j1 = j0 + 1
        pltpu.async_copy(table_hbm.at[idx_v.at[j1]], bufb, semb)
        pltpu.make_async_copy(table_hbm.at[idx_v.at[j0]], bufa, sema).wait()
        pltpu.sync_copy(bufa, out_hbm.at[pl.ds(out_base + j0 * _CHUNK, _CHUNK)])

        @pl.when(jj + 1 < _NCH // 2)
        def _():
            pltpu.async_copy(table_hbm.at[idx_v.at[j0 + 2]], bufa, sema)

        pltpu.make_async_copy(table_hbm.at[idx_v.at[j1]], bufb, semb).wait()
        pltpu.sync_copy(bufb, out_hbm.at[pl.ds(out_base + j1 * _CHUNK, _CHUNK)])
        return carry

    lax.fori_loop(0, _NCH // 2, step, 0)


@jax.jit
def _sc_gather(table, idx2d):
    mesh = plsc.VectorSubcoreMesh(core_axis_name="c", subcore_axis_name="s")
    return pl.kernel(
        _sc_gather_body,
        out_type=jax.ShapeDtypeStruct((_TOT, _D), jnp.float32),
        mesh=mesh,
        scratch_types=[
            pltpu.VMEM((_NCH, _CHUNK), jnp.int32),
            pltpu.VMEM((_CHUNK, _D), jnp.float32),
            pltpu.VMEM((_CHUNK, _D), jnp.float32),
            pltpu.SemaphoreType.DMA,
            pltpu.SemaphoreType.DMA,
        ],
    )(table, idx2d)


_BS = 512


def _tc_body(feat_ref, sid_ref, hw1, hb1, hw2, hb2, dmw, dmb,
             sw1, sb1, sw2, sb2, exw, exb, gw, gb, tww, twb, pdw, pdb, sgw,
             o0, o1, o2, om):
    x = feat_ref[...]
    sid = sid_ref[...]

    h = jnp.maximum(jnp.dot(x, hw1[...], preferred_element_type=jnp.float32)
                    + hb1[...], 0.0)
    h = jnp.maximum(jnp.dot(h, hw2[...], preferred_element_type=jnp.float32)
                    + hb2[...], 0.0)
    m = jnp.dot(h, dmw[...], preferred_element_type=jnp.float32) + dmb[...]
    s = jax.nn.sigmoid(m)
    oh = (lax.broadcasted_iota(jnp.int32, (1, 3), 1) == sid).astype(jnp.float32)
    ssel = jnp.sum(s * oh, axis=1, keepdims=True)
    om[...] = jnp.sign(jnp.maximum(ssel - 0.5, 0.0))

    sh = jnp.maximum(jnp.dot(x, sw1[...], preferred_element_type=jnp.float32)
                     + sb1[...], 0.0)
    sh = jnp.maximum(jnp.dot(sh, sw2[...], preferred_element_type=jnp.float32)
                     + sb2[...], 0.0)
    eo = jnp.maximum(jnp.dot(sh, exw[...], preferred_element_type=jnp.float32)
                     + exb[...], 0.0)
    gl = jnp.dot(sh, gw[...], preferred_element_type=jnp.float32) + gb[...]
    wlog = jnp.dot(sh, sgw[...], preferred_element_type=jnp.float32)

    ps = []
    for i in range(3):
        gi = gl[:, 8 * i:8 * (i + 1)]
        gi = gi - jnp.max(gi, axis=1, keepdims=True)
        egi = jnp.exp(gi)
        g = egi / jnp.sum(egi, axis=1, keepdims=True)
        mix = g[:, 0:1] * eo[:, 0:64]
        for e in range(1, 8):
            mix = mix + g[:, e:e + 1] * eo[:, 64 * e:64 * (e + 1)]
        t = jnp.maximum(
            jnp.dot(mix, tww[:, 64 * i:64 * (i + 1)],
                    preferred_element_type=jnp.float32) + twb[64 * i:64 * (i + 1)],
            0.0)
        p = jax.nn.sigmoid(
            jnp.dot(t, pdw[:, i:i + 1], preferred_element_type=jnp.float32)
            + pdb[:, i:i + 1])
        ps.append(p)
    sc = jnp.concatenate(ps, axis=1)

    for i, oref in enumerate((o0, o1, o2)):
        wi = wlog[:, 3 * i:3 * (i + 1)]
        wi = wi - jnp.max(wi, axis=1, keepdims=True)
        ew = jnp.exp(wi)
        w = ew / jnp.sum(ew, axis=1, keepdims=True)
        oref[...] = jnp.sum(w * sc, axis=1)


def _full(shape):
    nd = len(shape)
    return pl.BlockSpec(shape, lambda i, _nd=nd: (0,) * _nd)


@jax.jit
def _tc_dense(feat, sid, hw1, hb1, hw2, hb2, dmw, dmb,
              sw1, sb1, sw2, sb2, exw, exb, gw, gb, tww, twb, pdw, pdb, sgw):
    grid = _B // _BS
    return pl.pallas_call(
        _tc_body,
        grid=(grid,),
        in_specs=[
            pl.BlockSpec((_BS, _IN), lambda i: (i, 0)),
            pl.BlockSpec((_BS, 1), lambda i: (i, 0)),
            _full((_IN, 128)), _full((128,)),
            _full((128, 64)), _full((64,)),
            _full((64, 3)), _full((1, 3)),
            _full((_IN, 256)), _full((256,)),
            _full((256, 128)), _full((128,)),
            _full((128, 512)), _full((512,)),
            _full((128, 24)), _full((24,)),
            _full((64, 192)), _full((192,)),
            _full((64, 3)), _full((1, 3)),
            _full((128, 9)),
        ],
        out_specs=[
            pl.BlockSpec((_BS,), lambda i: (i,)),
            pl.BlockSpec((_BS,), lambda i: (i,)),
            pl.BlockSpec((_BS,), lambda i: (i,)),
            pl.BlockSpec((_BS, 1), lambda i: (i, 0)),
        ],
        out_shape=[
            jax.ShapeDtypeStruct((_B,), jnp.float32),
            jax.ShapeDtypeStruct((_B,), jnp.float32),
            jax.ShapeDtypeStruct((_B,), jnp.float32),
            jax.ShapeDtypeStruct((_B, 1), jnp.float32),
        ],
    )(feat, sid, hw1, hb1, hw2, hb2, dmw, dmb,
      sw1, sb1, sw2, sb2, exw, exb, gw, gb, tww, twb, pdw, pdb, sgw)


def kernel(inputs, label, sid, emb_table, dom_emb_table, share_W1, share_b1,
           share_W2, share_b2, expert_W, expert_b, gate_W, gate_b, tower_W,
           tower_b, pred_W, pred_b, sg_W, hyper_W1, hyper_b1, hyper_W2,
           hyper_b2, dm_W, dm_b):
    idx2d = inputs.astype(jnp.int32).reshape(_TOT // _CHUNK, _CHUNK)
    feat = _sc_gather(emb_table, idx2d).reshape(_B, _IN)

    dmw = dm_W[:, :, 0].T
    dmb = dm_b.reshape(1, 3)
    exw = expert_W.transpose(1, 0, 2).reshape(128, 512)
    exb = expert_b.reshape(512)
    gw = gate_W.transpose(1, 0, 2).reshape(128, 24)
    gb = gate_b.reshape(24)
    tww = tower_W.transpose(1, 0, 2).reshape(64, 192)
    twb = tower_b.reshape(192)
    pdw = pred_W[:, :, 0].T
    pdb = pred_b.reshape(1, 3)
    sgw = sg_W.transpose(1, 0, 2).reshape(128, 9)

    o0, o1, o2, smask = _tc_dense(
        feat, sid, hyper_W1, hyper_b1, hyper_W2, hyper_b2, dmw, dmb,
        share_W1, share_b1, share_W2, share_b2, exw, exb, gw, gb,
        tww, twb, pdw, pdb, sgw)
    return (o0, o1, o2, sid, label, smask)


# speedup vs baseline: 11.1402x; 11.1402x over previous
import functools

import jax
import jax.numpy as jnp
from jax import lax
from jax.experimental import pallas as pl
from jax.experimental.pallas import tpu as pltpu
from jax.experimental.pallas import tpu_sc as plsc

_B = 16384
_F = 26
_D = 16
_IN = _F * _D

_NC = 2
_NS = 16
_NW = _NC * _NS
_TOT = _B * _F
_RPW = _TOT // _NW
_CHUNK = 128
_NCH = _RPW // _CHUNK


def _sc_gather_body(table_hbm, idx_hbm, out_hbm, idx_v, bufa, bufb, sema, semb):
    wid = lax.axis_index("s") * _NC + lax.axis_index("c")
    pltpu.sync_copy(idx_hbm.at[pl.ds(wid * _NCH, _NCH)], idx_v)
    out_base = wid * _RPW
    pltpu.async_copy(table_hbm.at[idx_v.at[0]], bufa, sema)

    def step(jj, carry):
        j0 = 2 * jj
        j1 = j0 + 1
        pltpu.async_copy(table_hbm.at[idx_v.at[j1]], bufb, semb)
        pltpu.make_async_copy(table_hbm.at[idx_v.at[j0]], bufa, sema).wait()
        pltpu.sync_copy(bufa, out_hbm.at[pl.ds(out_base + j0 * _CHUNK, _CHUNK)])

        @pl.when(jj + 1 < _NCH // 2)
        def _():
            pltpu.async_copy(table_hbm.at[idx_v.at[j0 + 2]], bufa, sema)

        pltpu.make_async_copy(table_hbm.at[idx_v.at[j1]], bufb, semb).wait()
        pltpu.sync_copy(bufb, out_hbm.at[pl.ds(out_base + j1 * _CHUNK, _CHUNK)])
        return carry

    lax.fori_loop(0, _NCH // 2, step, 0)


@jax.jit
def _sc_gather(table, idx2d):
    mesh = plsc.VectorSubcoreMesh(core_axis_name="c", subcore_axis_name="s")
    return pl.kernel(
        _sc_gather_body,
        out_type=jax.ShapeDtypeStruct((_TOT, _D), jnp.float32),
        mesh=mesh,
        scratch_types=[
            pltpu.VMEM((_NCH, _CHUNK), jnp.int32),
            pltpu.VMEM((_CHUNK, _D), jnp.float32),
            pltpu.VMEM((_CHUNK, _D), jnp.float32),
            pltpu.SemaphoreType.DMA,
            pltpu.SemaphoreType.DMA,
        ],
        compiler_params=pltpu.CompilerParams(use_tc_tiling_on_sc=False),
    )(table, idx2d)


_BS = 512


def _tc_body(feat_ref, sid_ref, hw1, hb1, hw2, hb2, dmw, dmb,
             sw1, sb1, sw2, sb2, exw, exb, gw, gb, tww, twb, pdw, pdb, sgw,
             o0, o1, o2, om):
    x = feat_ref[...]
    sid = sid_ref[...]

    h = jnp.maximum(jnp.dot(x, hw1[...], preferred_element_type=jnp.float32)
                    + hb1[...], 0.0)
    h = jnp.maximum(jnp.dot(h, hw2[...], preferred_element_type=jnp.float32)
                    + hb2[...], 0.0)
    m = jnp.dot(h, dmw[...], preferred_element_type=jnp.float32) + dmb[...]
    s = jax.nn.sigmoid(m)
    oh = (lax.broadcasted_iota(jnp.int32, (1, 3), 1) == sid).astype(jnp.float32)
    ssel = jnp.sum(s * oh, axis=1, keepdims=True)
    om[...] = jnp.sign(jnp.maximum(ssel - 0.5, 0.0))

    sh = jnp.maximum(jnp.dot(x, sw1[...], preferred_element_type=jnp.float32)
                     + sb1[...], 0.0)
    sh = jnp.maximum(jnp.dot(sh, sw2[...], preferred_element_type=jnp.float32)
                     + sb2[...], 0.0)
    eo = jnp.maximum(jnp.dot(sh, exw[...], preferred_element_type=jnp.float32)
                     + exb[...], 0.0)
    gl = jnp.dot(sh, gw[...], preferred_element_type=jnp.float32) + gb[...]
    wlog = jnp.dot(sh, sgw[...], preferred_element_type=jnp.float32)

    ps = []
    for i in range(3):
        gi = gl[:, 8 * i:8 * (i + 1)]
        gi = gi - jnp.max(gi, axis=1, keepdims=True)
        egi = jnp.exp(gi)
        g = egi / jnp.sum(egi, axis=1, keepdims=True)
        mix = g[:, 0:1] * eo[:, 0:64]
        for e in range(1, 8):
            mix = mix + g[:, e:e + 1] * eo[:, 64 * e:64 * (e + 1)]
        t = jnp.maximum(
            jnp.dot(mix, tww[:, 64 * i:64 * (i + 1)],
                    preferred_element_type=jnp.float32) + twb[64 * i:64 * (i + 1)],
            0.0)
        p = jax.nn.sigmoid(
            jnp.dot(t, pdw[:, i:i + 1], preferred_element_type=jnp.float32)
            + pdb[:, i:i + 1])
        ps.append(p)
    sc = jnp.concatenate(ps, axis=1)

    for i, oref in enumerate((o0, o1, o2)):
        wi = wlog[:, 3 * i:3 * (i + 1)]
        wi = wi - jnp.max(wi, axis=1, keepdims=True)
        ew = jnp.exp(wi)
        w = ew / jnp.sum(ew, axis=1, keepdims=True)
        oref[...] = jnp.sum(w * sc, axis=1)


def _full(shape):
    nd = len(shape)
    return pl.BlockSpec(shape, lambda i, _nd=nd: (0,) * _nd)


@jax.jit
def _tc_dense(feat, sid, hw1, hb1, hw2, hb2, dmw, dmb,
              sw1, sb1, sw2, sb2, exw, exb, gw, gb, tww, twb, pdw, pdb, sgw):
    grid = _B // _BS
    return pl.pallas_call(
        _tc_body,
        grid=(grid,),
        in_specs=[
            pl.BlockSpec((_BS, _IN), lambda i: (i, 0)),
            pl.BlockSpec((_BS, 1), lambda i: (i, 0)),
            _full((_IN, 128)), _full((128,)),
            _full((128, 64)), _full((64,)),
            _full((64, 3)), _full((1, 3)),
            _full((_IN, 256)), _full((256,)),
            _full((256, 128)), _full((128,)),
            _full((128, 512)), _full((512,)),
            _full((128, 24)), _full((24,)),
            _full((64, 192)), _full((192,)),
            _full((64, 3)), _full((1, 3)),
            _full((128, 9)),
        ],
        out_specs=[
            pl.BlockSpec((_BS,), lambda i: (i,)),
            pl.BlockSpec((_BS,), lambda i: (i,)),
            pl.BlockSpec((_BS,), lambda i: (i,)),
            pl.BlockSpec((_BS, 1), lambda i: (i, 0)),
        ],
        out_shape=[
            jax.ShapeDtypeStruct((_B,), jnp.float32),
            jax.ShapeDtypeStruct((_B,), jnp.float32),
            jax.ShapeDtypeStruct((_B,), jnp.float32),
            jax.ShapeDtypeStruct((_B, 1), jnp.float32),
        ],
    )(feat, sid, hw1, hb1, hw2, hb2, dmw, dmb,
      sw1, sb1, sw2, sb2, exw, exb, gw, gb, tww, twb, pdw, pdb, sgw)


def kernel(inputs, label, sid, emb_table, dom_emb_table, share_W1, share_b1,
           share_W2, share_b2, expert_W, expert_b, gate_W, gate_b, tower_W,
           tower_b, pred_W, pred_b, sg_W, hyper_W1, hyper_b1, hyper_W2,
           hyper_b2, dm_W, dm_b):
    idx2d = inputs.astype(jnp.int32).reshape(_TOT // _CHUNK, _CHUNK)
    feat = _sc_gather(emb_table, idx2d).reshape(_B, _IN)

    dmw = dm_W[:, :, 0].T
    dmb = dm_b.reshape(1, 3)
    exw = expert_W.transpose(1, 0, 2).reshape(128, 512)
    exb = expert_b.reshape(512)
    gw = gate_W.transpose(1, 0, 2).reshape(128, 24)
    gb = gate_b.reshape(24)
    tww = tower_W.transpose(1, 0, 2).reshape(64, 192)
    twb = tower_b.reshape(192)
    pdw = pred_W[:, :, 0].T
    pdb = pred_b.reshape(1, 3)
    sgw = sg_W.transpose(1, 0, 2).reshape(128, 9)

    o0, o1, o2, smask = _tc_dense(
        feat, sid, hyper_W1, hyper_b1, hyper_W2, hyper_b2, dmw, dmb,
        share_W1, share_b1, share_W2, share_b2, exw, exb, gw, gb,
        tww, twb, pdw, pdb, sgw)
    return (o0, o1, o2, sid, label, smask)
